# baseline (speedup 1.0000x reference)
import jax
import jax.numpy as jnp
from jax import lax
from jax.experimental import pallas as pl
from jax.experimental.pallas import tpu as pltpu
from jax.experimental.pallas import tpu_sc as plsc

N = 10000
E = 320000
D = 128
DE = 16
G = 128
C = 10
BN_INV = 1.0 / (1.0 + 1e-5) ** 0.5

_NC = 2
_NS = 16
_DH = D // _NC
_CH = 128
_NCH = 160
_EPT = _CH * _NCH
_EPAD = _NS * _EPT
_NP = 10240
_RPS = _NP // _NS
_ZR = 128

_BE = 4000
_BN = 2000


def _ee_body(ea_ref, w1_ref, b1_ref, w2_ref, b2_ref, g_ref, bt_ref, out_ref):
    ea = ea_ref[...]
    t = lax.dot_general(ea, w1_ref[...], (((1,), (0,)), ((), ())),
                        preferred_element_type=jnp.float32)
    t = jnp.maximum(t + b1_ref[...], 0.0)
    t = lax.dot_general(t, w2_ref[...], (((1,), (0,)), ((), ())),
                        preferred_element_type=jnp.float32)
    t = jnp.maximum(t + b2_ref[...], 0.0)
    t = t * g_ref[...] + bt_ref[...]
    out_ref[0] = t[:, :_DH]
    out_ref[1] = t[:, _DH:]


_ee_call = pl.pallas_call(
    _ee_body,
    grid=(E // _BE,),
    in_specs=[
        pl.BlockSpec((_BE, DE), lambda i: (i, 0)),
        pl.BlockSpec((DE, D), lambda i: (0, 0)),
        pl.BlockSpec((1, D), lambda i: (0, 0)),
        pl.BlockSpec((D, D), lambda i: (0, 0)),
        pl.BlockSpec((1, D), lambda i: (0, 0)),
        pl.BlockSpec((1, D), lambda i: (0, 0)),
        pl.BlockSpec((1, D), lambda i: (0, 0)),
    ],
    out_specs=pl.BlockSpec((_NC, _BE, _DH), lambda i: (0, i, 0)),
    out_shape=jax.ShapeDtypeStruct((_NC, _EPAD, _DH), jnp.float32),
)


def _sc_body(h_hbm, e_hbm, sd_hbm, out_hbm,
             ebuf, hbuf, ibuf, agg_sh, esem, gsem, ssem, isem):
    cid = lax.axis_index("c")
    sid = lax.axis_index("s")
    h_t = h_hbm.at[cid]
    e_t = e_hbm.at[cid]
    o_t = out_hbm.at[cid]
    crow = sid * _NCH

    def _idx_dma(c, s8):
        return pltpu.make_async_copy(sd_hbm.at[crow + c], ibuf.at[s8],
                                     isem.at[s8])

    def _e_dma(c, b):
        eb = (sid * _EPT + c * _CH,)
        return pltpu.make_async_copy(e_t.at[pl.ds(eb[0], _CH)], ebuf.at[b],
                                     esem.at[b])

    def _g_dma(c, b, s8):
        return pltpu.make_async_copy(h_t.at[ibuf.at[s8, 0]], hbuf.at[b],
                                     gsem.at[b])

    def _s_start(b, s8):
        pltpu.async_copy(ebuf.at[b], agg_sh.at[ibuf.at[s8, 1]],
                         ssem.at[b], add=True)

    def _s_wait(b, s8):
        pltpu.make_async_copy(ebuf.at[b], agg_sh.at[ibuf.at[s8, 1]],
                              ssem.at[b]).wait()

    def _zrow(k, carry):
        for d2 in range(_DH // 16):
            ebuf[0, k, pl.ds(d2 * 16, 16)] = jnp.zeros((16,), jnp.float32)
        return carry
    lax.fori_loop(0, _ZR, _zrow, 0)
    row0 = sid * _RPS
    for j in range(_RPS // _ZR):
        pltpu.sync_copy(ebuf.at[0], agg_sh.at[pl.ds(row0 + j * _ZR, _ZR)])

    for c in range(3):
        _idx_dma(c, c).start()
    for c in range(2):
        _idx_dma(c, c).wait()
        _e_dma(c, c).start()
        _g_dma(c, c, c).start()
    plsc.subcore_barrier()

    def _group(g, carry):
        c0 = g * 8
        for u in range(8):
            c = c0 + u
            b = u % 4
            _e_dma(c, b).wait()
            _g_dma(c, b, u).wait()

            def _rows(k4, inner, _b=b):
                r = k4 * 4
                for rr in range(4):
                    for d2 in range(_DH // 16):
                        sl = pl.ds(d2 * 16, 16)
                        ebuf[_b, r + rr, sl] = jnp.maximum(
                            ebuf[_b, r + rr, sl] + hbuf[_b, r + rr, sl], 0.0)
                return inner
            lax.fori_loop(0, _CH // 4, _rows, 0)
            _s_start(b, u % 8)

            b2 = (u + 2) % 4
            s8i = (u + 3) % 8
            s8g = (u + 2) % 8

            @pl.when(c + 3 < _NCH)
            def _():
                _idx_dma(c + 3, s8i).start()

            @pl.when(c + 2 < _NCH)
            def _():
                @pl.when(c >= 2)
                def _():
                    _s_wait(b2, (u + 6) % 8)
                _e_dma(c + 2, b2).start()
                _idx_dma(c + 2, s8g).wait()
                _g_dma(c + 2, b2, s8g).start()
        return carry
    lax.fori_loop(0, _NCH // 8, _group, 0)

    for u in range(4):
        _s_wait(u, (_NCH - 4 + u) % 8)
    plsc.subcore_barrier()
    for j in range(_RPS // _ZR):
        r = row0 + j * _ZR
        pltpu.sync_copy(agg_sh.at[pl.ds(r, _ZR)], ebuf.at[0])
        pltpu.sync_copy(ebuf.at[0], o_t.at[pl.ds(r, _ZR)])


_sc_call = pl.kernel(
    _sc_body,
    out_type=jax.ShapeDtypeStruct((_NC, _NP, _DH), jnp.float32),
    mesh=plsc.VectorSubcoreMesh(core_axis_name="c", subcore_axis_name="s",
                                num_cores=_NC, num_subcores=_NS),
    scratch_types=[
        pltpu.VMEM((4, _CH, _DH), jnp.float32),
        pltpu.VMEM((4, _CH, _DH), jnp.float32),
        pltpu.VMEM((8, 2, _CH), jnp.int32),
        pltpu.VMEM_SHARED((_NP, _DH), jnp.float32),
        pltpu.SemaphoreType.DMA((4,)),
        pltpu.SemaphoreType.DMA((4,)),
        pltpu.SemaphoreType.DMA((4,)),
        pltpu.SemaphoreType.DMA((8,)),
    ],
    compiler_params=pltpu.CompilerParams(use_tc_tiling_on_sc=False),
)


def _mlp_body(h_ref, agg_ref, w1_ref, b1_ref, w2_ref, b2_ref, g_ref, bt_ref,
              out_ref, out2_ref):
    t = h_ref[...] + jnp.concatenate([agg_ref[0], agg_ref[1]], axis=1)
    t = lax.dot_general(t, w1_ref[...], (((1,), (0,)), ((), ())),
                        preferred_element_type=jnp.float32)
    t = jnp.maximum(t + b1_ref[...], 0.0)
    t = lax.dot_general(t, w2_ref[...], (((1,), (0,)), ((), ())),
                        preferred_element_type=jnp.float32)
    t = jnp.maximum(t + b2_ref[...], 0.0)
    t = t * g_ref[...] + bt_ref[...]
    out_ref[...] = t
    out2_ref[0] = t[:, :_DH]
    out2_ref[1] = t[:, _DH:]


_mlp_call = pl.pallas_call(
    _mlp_body,
    grid=(N // _BN,),
    in_specs=[
        pl.BlockSpec((_BN, D), lambda i: (i, 0)),
        pl.BlockSpec((_NC, _BN, _DH), lambda i: (0, i, 0)),
        pl.BlockSpec((D, D), lambda i: (0, 0)),
        pl.BlockSpec((1, D), lambda i: (0, 0)),
        pl.BlockSpec((D, D), lambda i: (0, 0)),
        pl.BlockSpec((1, D), lambda i: (0, 0)),
        pl.BlockSpec((1, D), lambda i: (0, 0)),
        pl.BlockSpec((1, D), lambda i: (0, 0)),
    ],
    out_specs=(
        pl.BlockSpec((_BN, D), lambda i: (i, 0)),
        pl.BlockSpec((_NC, _BN, _DH), lambda i: (0, i, 0)),
    ),
    out_shape=(
        jax.ShapeDtypeStruct((N, D), jnp.float32),
        jax.ShapeDtypeStruct((_NC, N, _DH), jnp.float32),
    ),
)


def _pool_body(b_ref, h_ref, l1w_ref, l1b_ref, l2w_ref, l2b_ref, out_ref,
               acc_ref, cnt_ref):
    i = pl.program_id(0)

    @pl.when(i == 0)
    def _init():
        acc_ref[...] = jnp.zeros_like(acc_ref)
        cnt_ref[...] = jnp.zeros_like(cnt_ref)

    onehot = (b_ref[...] == lax.broadcasted_iota(jnp.int32, (_BN, G), 1)
              ).astype(jnp.float32)
    acc_ref[...] += lax.dot_general(onehot, h_ref[...], (((0,), (0,)), ((), ())),
                                    preferred_element_type=jnp.float32)
    cnt_ref[...] += lax.dot_general(onehot, jnp.ones((_BN, 1), jnp.float32),
                                    (((0,), (0,)), ((), ())),
                                    preferred_element_type=jnp.float32)

    @pl.when(i == pl.num_programs(0) - 1)
    def _fin():
        pooled = acc_ref[...] / jnp.maximum(cnt_ref[...], 1.0)
        o = lax.dot_general(pooled, l1w_ref[...], (((1,), (0,)), ((), ())),
                            preferred_element_type=jnp.float32)
        o = jnp.maximum(o + l1b_ref[...], 0.0)
        logits = lax.dot_general(o, l2w_ref[...], (((1,), (0,)), ((), ())),
                                 preferred_element_type=jnp.float32)
        logits = logits + l2b_ref[...]
        m = jnp.max(logits, axis=1, keepdims=True)
        lse = m + jnp.log(jnp.sum(jnp.exp(logits - m), axis=1, keepdims=True))
        out_ref[...] = logits - lse


_pool_call = pl.pallas_call(
    _pool_body,
    grid=(N // _BN,),
    in_specs=[
        pl.BlockSpec((_BN, 1), lambda i: (i, 0)),
        pl.BlockSpec((_BN, D), lambda i: (i, 0)),
        pl.BlockSpec((D, D), lambda i: (0, 0)),
        pl.BlockSpec((1, D), lambda i: (0, 0)),
        pl.BlockSpec((D, C), lambda i: (0, 0)),
        pl.BlockSpec((1, C), lambda i: (0, 0)),
    ],
    out_specs=pl.BlockSpec((G, C), lambda i: (0, 0)),
    out_shape=jax.ShapeDtypeStruct((G, C), jnp.float32),
    scratch_shapes=[
        pltpu.VMEM((G, D), jnp.float32),
        pltpu.VMEM((G, 1), jnp.float32),
    ],
)


def kernel(x, edge_index, edge_attr, batch,
           eW1, eb1, eW2, eb2, eg, ebt,
           mW1, mb1, mW2, mb2, mg, mbt,
           lin1_W, lin1_b, lin2_W, lin2_b):
    src = edge_index[0].astype(jnp.int32)
    dst = edge_index[1].astype(jnp.int32)
    pad = _EPAD - E
    srcp = jnp.concatenate([src, jnp.zeros((pad,), jnp.int32)])
    dstp = jnp.concatenate([dst, jnp.full((pad,), _NP - 1, jnp.int32)])
    sd = jnp.stack([srcp.reshape(_NS, _NCH, _CH),
                    dstp.reshape(_NS, _NCH, _CH)], axis=2)
    sd = sd.reshape(_NS * _NCH, 2, _CH)
    batch2 = batch.astype(jnp.int32).reshape(N, 1)
    eg_s = (eg * BN_INV).reshape(3, 1, D)
    ebt2 = ebt.reshape(3, 1, D)
    mg_s = (mg * BN_INV).reshape(3, 1, D)
    mbt2 = mbt.reshape(3, 1, D)
    eb1_2 = eb1.reshape(3, 1, D)
    eb2_2 = eb2.reshape(3, 1, D)
    mb1_2 = mb1.reshape(3, 1, D)
    mb2_2 = mb2.reshape(3, 1, D)

    h = x
    h2 = jnp.stack([x[:, :_DH], x[:, _DH:]])
    for l in range(3):
        e2 = _ee_call(edge_attr, eW1[l], eb1_2[l], eW2[l], eb2_2[l],
                      eg_s[l], ebt2[l])
        agg2 = _sc_call(h2, e2, sd)
        h, h2 = _mlp_call(h, agg2, mW1[l], mb1_2[l],
                          mW2[l], mb2_2[l], mg_s[l], mbt2[l])
    return _pool_call(batch2, h, lin1_W, lin1_b.reshape(1, D),
                      lin2_W, lin2_b.reshape(1, C))


# speedup vs baseline: 1.1571x; 1.1571x over previous
import jax
import jax.numpy as jnp
from jax import lax
from jax.experimental import pallas as pl
from jax.experimental.pallas import tpu as pltpu
from jax.experimental.pallas import tpu_sc as plsc

N = 10000
E = 320000
D = 128
DE = 16
G = 128
C = 10
BN_INV = 1.0 / (1.0 + 1e-5) ** 0.5

_NC = 2
_NS = 16
_TILES = _NC * _NS
_CH = 64
_NCH = 160
_EPT = _CH * _NCH
_EPAD = _TILES * _EPT
_NP = 10016
_RPS = 624

_BE = 4000
_BN = 2000


def _ee_body(ea_ref, w1_ref, b1_ref, w2_ref, b2_ref, g_ref, bt_ref, out_ref):
    ea = ea_ref[...]
    t = lax.dot_general(ea, w1_ref[...], (((1,), (0,)), ((), ())),
                        preferred_element_type=jnp.float32)
    t = jnp.maximum(t + b1_ref[...], 0.0)
    t = lax.dot_general(t, w2_ref[...], (((1,), (0,)), ((), ())),
                        preferred_element_type=jnp.float32)
    t = jnp.maximum(t + b2_ref[...], 0.0)
    out_ref[...] = t * g_ref[...] + bt_ref[...]


_ee_call = pl.pallas_call(
    _ee_body,
    grid=(E // _BE,),
    in_specs=[
        pl.BlockSpec((_BE, DE), lambda i: (i, 0)),
        pl.BlockSpec((DE, D), lambda i: (0, 0)),
        pl.BlockSpec((1, D), lambda i: (0, 0)),
        pl.BlockSpec((D, D), lambda i: (0, 0)),
        pl.BlockSpec((1, D), lambda i: (0, 0)),
        pl.BlockSpec((1, D), lambda i: (0, 0)),
        pl.BlockSpec((1, D), lambda i: (0, 0)),
    ],
    out_specs=pl.BlockSpec((_BE, D), lambda i: (i, 0)),
    out_shape=jax.ShapeDtypeStruct((_EPAD, D), jnp.float32),
)


def _sc_body(h_hbm, e_hbm, src_hbm, dst_hbm, out_hbm,
             ebuf, hbuf, mbuf, sibuf, dibuf, agg_sh,
             esem, gsem, ssem, sisem, disem):
    cid = lax.axis_index("c")
    sid = lax.axis_index("s")
    wid = cid * _NS + sid
    base = wid * _EPT

    def _si_dma(c):
        return pltpu.make_async_copy(src_hbm.at[pl.ds(base + c * _CH, _CH)],
                                     sibuf.at[c % 4], sisem.at[c % 4])

    def _di_dma(c):
        return pltpu.make_async_copy(dst_hbm.at[pl.ds(base + c * _CH, _CH)],
                                     dibuf.at[c % 8], disem.at[c % 8])

    def _e_dma(c, b):
        return pltpu.make_async_copy(e_hbm.at[pl.ds(base + c * _CH, _CH)],
                                     ebuf.at[b], esem.at[b])

    def _g_dma(c, b):
        return pltpu.make_async_copy(h_hbm.at[sibuf.at[c % 4]], hbuf.at[b],
                                     gsem.at[b])

    def _s_start(c, m):
        pltpu.async_copy(mbuf.at[m], agg_sh.at[dibuf.at[c % 8]],
                         ssem.at[m], add=True)

    def _s_wait(c, m):
        pltpu.make_async_copy(mbuf.at[m], agg_sh.at[dibuf.at[c % 8]],
                              ssem.at[m]).wait()

    def _zrow(k, carry):
        for d2 in range(D // 16):
            mbuf[0, k, pl.ds(d2 * 16, 16)] = jnp.zeros((16,), jnp.float32)
        return carry
    lax.fori_loop(0, _CH, _zrow, 0)
    row0 = sid * _RPS
    for r0, rn in ((0, 64), (64, 64), (128, 64), (192, 64), (256, 64),
                   (320, 64), (384, 64), (448, 64), (512, 64), (576, 48)):
        pltpu.sync_copy(mbuf.at[0, pl.ds(0, rn)],
                        agg_sh.at[pl.ds(row0 + r0, rn)])

    @pl.when(sid == _NS - 1)
    def _ztail():
        pltpu.sync_copy(mbuf.at[0, pl.ds(0, 32)],
                        agg_sh.at[pl.ds(_NP - 32, 32)])

    for c in range(3):
        _si_dma(c).start()
        _di_dma(c).start()
    for c in range(2):
        _si_dma(c).wait()
        _e_dma(c, c).start()
        _g_dma(c, c).start()
    plsc.subcore_barrier()

    def _group(g, carry):
        c0 = g * 8
        for u in range(8):
            c = c0 + u
            b = u % 2
            _e_dma(c, b).wait()
            _g_dma(c, b).wait()

            @pl.when(c >= 2)
            def _():
                _s_wait(c - 2, b)

            def _rows(k4, inner, _b=b):
                r = k4 * 4
                for rr in range(4):
                    for d2 in range(D // 16):
                        sl = pl.ds(d2 * 16, 16)
                        mbuf[_b, r + rr, sl] = jnp.maximum(
                            ebuf[_b, r + rr, sl] + hbuf[_b, r + rr, sl], 0.0)
                return inner
            lax.fori_loop(0, _CH // 4, _rows, 0)
            _s_start(c, b)

            @pl.when(c + 3 < _NCH)
            def _():
                _si_dma(c + 3).start()
                _di_dma(c + 3).start()

            @pl.when(c + 2 < _NCH)
            def _():
                _e_dma(c + 2, b).start()
                _si_dma(c + 2).wait()
                _g_dma(c + 2, b).start()
        return carry
    lax.fori_loop(0, _NCH // 8, _group, 0)

    _s_wait(_NCH - 2, 0)
    _s_wait(_NCH - 1, 1)
    plsc.subcore_barrier()
    for r0, rn in ((0, 64), (64, 64), (128, 64), (192, 64), (256, 64),
                   (320, 64), (384, 64), (448, 64), (512, 64), (576, 48)):
        pltpu.sync_copy(agg_sh.at[pl.ds(row0 + r0, rn)],
                        mbuf.at[0, pl.ds(0, rn)])
        pltpu.sync_copy(mbuf.at[0, pl.ds(0, rn)],
                        out_hbm.at[pl.ds(cid * _NP + row0 + r0, rn)])

    @pl.when(sid == _NS - 1)
    def _ftail():
        pltpu.sync_copy(agg_sh.at[pl.ds(_NP - 32, 32)],
                        mbuf.at[0, pl.ds(0, 32)])
        pltpu.sync_copy(mbuf.at[0, pl.ds(0, 32)],
                        out_hbm.at[pl.ds(cid * _NP + _NP - 32, 32)])


_sc_call = pl.kernel(
    _sc_body,
    out_type=jax.ShapeDtypeStruct((_NC * _NP, D), jnp.float32),
    mesh=plsc.VectorSubcoreMesh(core_axis_name="c", subcore_axis_name="s",
                                num_cores=_NC, num_subcores=_NS),
    scratch_types=[
        pltpu.VMEM((2, _CH, D), jnp.float32),
        pltpu.VMEM((2, _CH, D), jnp.float32),
        pltpu.VMEM((2, _CH, D), jnp.float32),
        pltpu.VMEM((4, _CH), jnp.int32),
        pltpu.VMEM((8, _CH), jnp.int32),
        pltpu.VMEM_SHARED((_NP, D), jnp.float32),
        pltpu.SemaphoreType.DMA((2,)),
        pltpu.SemaphoreType.DMA((2,)),
        pltpu.SemaphoreType.DMA((2,)),
        pltpu.SemaphoreType.DMA((4,)),
        pltpu.SemaphoreType.DMA((8,)),
    ],
)


def _mlp_body(h_ref, agg_ref, w1_ref, b1_ref, w2_ref, b2_ref, g_ref, bt_ref,
              out_ref):
    t = h_ref[...] + agg_ref[0] + agg_ref[1]
    t = lax.dot_general(t, w1_ref[...], (((1,), (0,)), ((), ())),
                        preferred_element_type=jnp.float32)
    t = jnp.maximum(t + b1_ref[...], 0.0)
    t = lax.dot_general(t, w2_ref[...], (((1,), (0,)), ((), ())),
                        preferred_element_type=jnp.float32)
    t = jnp.maximum(t + b2_ref[...], 0.0)
    out_ref[...] = t * g_ref[...] + bt_ref[...]


_mlp_call = pl.pallas_call(
    _mlp_body,
    grid=(N // _BN,),
    in_specs=[
        pl.BlockSpec((_BN, D), lambda i: (i, 0)),
        pl.BlockSpec((_NC, _BN, D), lambda i: (0, i, 0)),
        pl.BlockSpec((D, D), lambda i: (0, 0)),
        pl.BlockSpec((1, D), lambda i: (0, 0)),
        pl.BlockSpec((D, D), lambda i: (0, 0)),
        pl.BlockSpec((1, D), lambda i: (0, 0)),
        pl.BlockSpec((1, D), lambda i: (0, 0)),
        pl.BlockSpec((1, D), lambda i: (0, 0)),
    ],
    out_specs=pl.BlockSpec((_BN, D), lambda i: (i, 0)),
    out_shape=jax.ShapeDtypeStruct((N, D), jnp.float32),
)


def _pool_body(b_ref, h_ref, l1w_ref, l1b_ref, l2w_ref, l2b_ref, out_ref,
               acc_ref, cnt_ref):
    i = pl.program_id(0)

    @pl.when(i == 0)
    def _init():
        acc_ref[...] = jnp.zeros_like(acc_ref)
        cnt_ref[...] = jnp.zeros_like(cnt_ref)

    onehot = (b_ref[...] == lax.broadcasted_iota(jnp.int32, (_BN, G), 1)
              ).astype(jnp.float32)
    acc_ref[...] += lax.dot_general(onehot, h_ref[...], (((0,), (0,)), ((), ())),
                                    preferred_element_type=jnp.float32)
    cnt_ref[...] += lax.dot_general(onehot, jnp.ones((_BN, 1), jnp.float32),
                                    (((0,), (0,)), ((), ())),
                                    preferred_element_type=jnp.float32)

    @pl.when(i == pl.num_programs(0) - 1)
    def _fin():
        pooled = acc_ref[...] / jnp.maximum(cnt_ref[...], 1.0)
        o = lax.dot_general(pooled, l1w_ref[...], (((1,), (0,)), ((), ())),
                            preferred_element_type=jnp.float32)
        o = jnp.maximum(o + l1b_ref[...], 0.0)
        logits = lax.dot_general(o, l2w_ref[...], (((1,), (0,)), ((), ())),
                                 preferred_element_type=jnp.float32)
        logits = logits + l2b_ref[...]
        m = jnp.max(logits, axis=1, keepdims=True)
        lse = m + jnp.log(jnp.sum(jnp.exp(logits - m), axis=1, keepdims=True))
        out_ref[...] = logits - lse


_pool_call = pl.pallas_call(
    _pool_body,
    grid=(N // _BN,),
    in_specs=[
        pl.BlockSpec((_BN, 1), lambda i: (i, 0)),
        pl.BlockSpec((_BN, D), lambda i: (i, 0)),
        pl.BlockSpec((D, D), lambda i: (0, 0)),
        pl.BlockSpec((1, D), lambda i: (0, 0)),
        pl.BlockSpec((D, C), lambda i: (0, 0)),
        pl.BlockSpec((1, C), lambda i: (0, 0)),
    ],
    out_specs=pl.BlockSpec((G, C), lambda i: (0, 0)),
    out_shape=jax.ShapeDtypeStruct((G, C), jnp.float32),
    scratch_shapes=[
        pltpu.VMEM((G, D), jnp.float32),
        pltpu.VMEM((G, 1), jnp.float32),
    ],
)


def kernel(x, edge_index, edge_attr, batch,
           eW1, eb1, eW2, eb2, eg, ebt,
           mW1, mb1, mW2, mb2, mg, mbt,
           lin1_W, lin1_b, lin2_W, lin2_b):
    src = edge_index[0].astype(jnp.int32)
    dst = edge_index[1].astype(jnp.int32)
    pad = _EPAD - E
    srcp = jnp.concatenate([src, jnp.zeros((pad,), jnp.int32)])
    dstp = jnp.concatenate([dst, jnp.full((pad,), _NP - 1, jnp.int32)])
    batch2 = batch.astype(jnp.int32).reshape(N, 1)
    eg_s = (eg * BN_INV).reshape(3, 1, D)
    ebt2 = ebt.reshape(3, 1, D)
    mg_s = (mg * BN_INV).reshape(3, 1, D)
    mbt2 = mbt.reshape(3, 1, D)
    eb1_2 = eb1.reshape(3, 1, D)
    eb2_2 = eb2.reshape(3, 1, D)
    mb1_2 = mb1.reshape(3, 1, D)
    mb2_2 = mb2.reshape(3, 1, D)

    h = x
    for l in range(3):
        e = _ee_call(edge_attr, eW1[l], eb1_2[l], eW2[l], eb2_2[l],
                     eg_s[l], ebt2[l])
        parts = _sc_call(h, e, srcp, dstp)
        h = _mlp_call(h, parts.reshape(_NC, _NP, D), mW1[l], mb1_2[l],
                      mW2[l], mb2_2[l], mg_s[l], mbt2[l])
    return _pool_call(batch2, h, lin1_W, lin1_b.reshape(1, D),
                      lin2_W, lin2_b.reshape(1, C))
